# baseline (speedup 1.0000x reference)
import jax
import jax.numpy as jnp
from jax import lax
from jax.experimental import pallas as pl
from jax.experimental.pallas import tpu as pltpu
from jax.experimental.pallas import tpu_sc as plsc

USER_N = 40000
ITEM_N = 60000
NN = USER_N + ITEM_N
NPAD = 102400
DD = 64
RR = 5
EE = 320000
NC = 2
NS = 16
EPT = EE // NS
CHUNK = 6400
KPC = NPAD // CHUNK // NC
STRIPE = CHUNK // NS
ZROWS = 80
BATCH = 128


def _spmm_body(*refs):
    dsts = refs[0:RR]
    srcs = refs[RR:2 * RR]
    vals = refs[2 * RR:3 * RR]
    x_hbm = refs[3 * RR]
    y_hbm = refs[3 * RR + 1]
    (dst_v, src_v, val_v, eidx_v,
     rows_a, gidx_a, sidx_a, vstg_a, sem_a, ssem_a,
     rows_b, gidx_b, sidx_b, vstg_b, sem_b, ssem_b,
     zidx_v, cnt_ref, acc_sh) = refs[3 * RR + 2:]

    core = lax.axis_index("c")
    tile = lax.axis_index("s")
    ebase = tile * EPT

    for r in range(RR):
        pltpu.sync_copy(dsts[r].at[pl.ds(ebase, EPT)], dst_v)
        pltpu.sync_copy(srcs[r].at[pl.ds(ebase, EPT)], src_v)
        pltpu.sync_copy(vals[r].at[pl.ds(ebase, EPT)], val_v)

        def chunk_body(k, carry, r=r):
            lo = (core * KPC + k) * CHUNK

            def zrow(i, c):
                for j in range(DD // 16):
                    rows_a[i, pl.ds(j * 16, 16)] = jnp.zeros((16,), jnp.float32)
                return c
            lax.fori_loop(0, ZROWS, zrow, 0)

            def zblk(z, c):
                zb = tile * STRIPE + z * ZROWS
                for j in range(ZROWS // 16):
                    zidx_v[pl.ds(j * 16, 16)] = (
                        zb + j * 16 + lax.iota(jnp.int32, 16))
                pltpu.sync_copy(rows_a.at[pl.ds(0, ZROWS)], acc_sh.at[zidx_v])
                return c
            lax.fori_loop(0, STRIPE // ZROWS, zblk, 0)
            plsc.subcore_barrier()

            cnt_ref[pl.ds(0, 16)] = jnp.zeros((16,), jnp.int32)

            def filt(i, c):
                d16 = dst_v[pl.ds(i * 16, 16)]
                m = (d16 >= lo) & (d16 < lo + CHUNK)
                e16 = lax.iota(jnp.int32, 16) + i * 16
                csum = plsc.cumsum(m.astype(jnp.int32))
                cntv = cnt_ref[pl.ds(0, 16)]
                plsc.store_scatter(eidx_v, [cntv + csum - 1], e16, mask=m)
                cnt_ref[pl.ds(0, 16)] = (
                    cntv + jnp.broadcast_to(csum[15], (16,)))
                return c
            lax.fori_loop(0, EPT // 16, filt, 0)

            cntv = cnt_ref[pl.ds(0, 16)]
            nb = ((cntv + (BATCH - 1)) // BATCH)[0]

            def stage(b, gidx_X, sidx_X, vstg_X):
                base = b * BATCH
                for j in range(BATCH // 16):
                    pos = base + j * 16 + lax.iota(jnp.int32, 16)
                    valid = pos < cnt_ref[pl.ds(0, 16)]
                    e16 = plsc.load_gather(eidx_v, [jnp.where(valid, pos, 0)])
                    e16 = jnp.where(valid, e16, 0)
                    s16 = plsc.load_gather(src_v, [e16])
                    d16 = plsc.load_gather(dst_v, [e16]) - lo
                    v16 = plsc.load_gather(val_v, [e16])
                    gidx_X[pl.ds(j * 16, 16)] = jnp.where(valid, s16, 0)
                    sidx_X[pl.ds(j * 16, 16)] = jnp.where(valid, d16, 0)
                    vstg_X[pl.ds(j * 16, 16)] = jnp.where(valid, v16, 0.0)

            def gather(gidx_X, rows_X, sem_X, r=r):
                return pltpu.make_async_copy(
                    x_hbm.at[r].at[gidx_X], rows_X, sem_X)

            def scatter(rows_X, sidx_X, ssem_X):
                return pltpu.make_async_copy(
                    rows_X, acc_sh.at[sidx_X], ssem_X)

            def consume(rows_X, sidx_X, vstg_X, ssem_X):
                def scale(g, cc):
                    v16 = vstg_X[pl.ds(g * 16, 16)]
                    for l in range(16):
                        vb = jnp.broadcast_to(v16[l], (16,))
                        i = g * 16 + l
                        for j in range(DD // 16):
                            rows_X[i, pl.ds(j * 16, 16)] = (
                                rows_X[i, pl.ds(j * 16, 16)] * vb)
                    return cc
                lax.fori_loop(0, BATCH // 16, scale, 0)
                scatter(rows_X, sidx_X, ssem_X).start(add=True)

            @pl.when(nb > 0)
            def _prologue():
                stage(0, gidx_a, sidx_a, vstg_a)
                gather(gidx_a, rows_a, sem_a).start()

            def pair_body(p, c):
                b0 = 2 * p

                @pl.when(b0 + 1 < nb)
                def _prefetch_b():
                    @pl.when(p > 0)
                    def _drain_b():
                        scatter(rows_b, sidx_b, ssem_b).wait()
                    stage(b0 + 1, gidx_b, sidx_b, vstg_b)
                    gather(gidx_b, rows_b, sem_b).start()

                gather(gidx_a, rows_a, sem_a).wait()
                consume(rows_a, sidx_a, vstg_a, ssem_a)

                @pl.when(b0 + 1 < nb)
                def _odd_batch():
                    @pl.when(b0 + 2 < nb)
                    def _prefetch_a():
                        scatter(rows_a, sidx_a, ssem_a).wait()
                        stage(b0 + 2, gidx_a, sidx_a, vstg_a)
                        gather(gidx_a, rows_a, sem_a).start()

                    gather(gidx_b, rows_b, sem_b).wait()
                    consume(rows_b, sidx_b, vstg_b, ssem_b)
                return c
            lax.fori_loop(0, (nb + 1) // 2, pair_body, 0)

            @pl.when(nb > 0)
            def _drain_last_a():
                scatter(rows_a, sidx_a, ssem_a).wait()

            @pl.when(nb >= 2)
            def _drain_last_b():
                scatter(rows_b, sidx_b, ssem_b).wait()
            plsc.subcore_barrier()

            def wblk(z, c, r=r):
                off = tile * STRIPE + z * ZROWS
                for j in range(ZROWS // 16):
                    zidx_v[pl.ds(j * 16, 16)] = (
                        off + j * 16 + lax.iota(jnp.int32, 16))
                pltpu.sync_copy(acc_sh.at[zidx_v], rows_a.at[pl.ds(0, ZROWS)])
                pltpu.sync_copy(rows_a.at[pl.ds(0, ZROWS)],
                                y_hbm.at[r].at[pl.ds(lo + off, ZROWS)])
                return c
            lax.fori_loop(0, STRIPE // ZROWS, wblk, 0)
            return carry
        lax.fori_loop(0, KPC, chunk_body, 0)


@jax.jit
def _spmm(dsts, srcs, vals, x):
    mesh = plsc.VectorSubcoreMesh(core_axis_name="c", subcore_axis_name="s",
                                  num_cores=NC, num_subcores=NS)
    return pl.kernel(
        _spmm_body,
        out_type=jax.ShapeDtypeStruct((RR, NPAD, DD), jnp.float32),
        mesh=mesh,
        compiler_params=pltpu.CompilerParams(
            needs_layout_passes=False, use_tc_tiling_on_sc=False),
        scratch_types=[
            pltpu.VMEM((EPT,), jnp.int32),
            pltpu.VMEM((EPT,), jnp.int32),
            pltpu.VMEM((EPT,), jnp.float32),
            pltpu.VMEM((EPT,), jnp.int32),
            pltpu.VMEM((BATCH, DD), jnp.float32),
            pltpu.VMEM((BATCH,), jnp.int32),
            pltpu.VMEM((BATCH,), jnp.int32),
            pltpu.VMEM((BATCH,), jnp.float32),
            pltpu.SemaphoreType.DMA,
            pltpu.SemaphoreType.DMA,
            pltpu.VMEM((BATCH, DD), jnp.float32),
            pltpu.VMEM((BATCH,), jnp.int32),
            pltpu.VMEM((BATCH,), jnp.int32),
            pltpu.VMEM((BATCH,), jnp.float32),
            pltpu.SemaphoreType.DMA,
            pltpu.SemaphoreType.DMA,
            pltpu.VMEM((ZROWS,), jnp.int32),
            pltpu.VMEM((16,), jnp.int32),
            pltpu.VMEM_SHARED((CHUNK, DD), jnp.float32),
        ],
    )(*dsts, *srcs, *vals, x)


BM = 2000
UBLOCKS = USER_N // BM


def _mix_body(y1_ref, y2_ref, um_ref, uv_ref, im_ref, iv_ref, o_ref):
    i = pl.program_id(0)
    is_user = i < UBLOCKS
    mat = jnp.where(is_user, um_ref[...], im_ref[...])
    vec = jnp.where(is_user, uv_ref[...], iv_ref[...])
    ms, ws = [], []
    for r in range(RR):
        m = (y1_ref[r] + y2_ref[r]) * 0.5
        h = jnp.dot(m, mat, preferred_element_type=jnp.float32)
        w = jnp.tanh(jnp.dot(h, vec, preferred_element_type=jnp.float32))
        ms.append(m)
        ws.append(w)
    wmax = ws[0]
    for r in range(1, RR):
        wmax = jnp.maximum(wmax, ws[r])
    es = [jnp.exp(w - wmax) for w in ws]
    denom = es[0]
    for r in range(1, RR):
        denom = denom + es[r]
    acc = ms[0] * (es[0] / denom)
    for r in range(1, RR):
        acc = acc + ms[r] * (es[r] / denom)
    o_ref[...] = acc


@jax.jit
def _mix(y1, y2, um, uv, im, iv):
    return pl.pallas_call(
        _mix_body,
        grid=(NN // BM,),
        in_specs=[
            pl.BlockSpec((RR, BM, DD), lambda i: (0, i, 0)),
            pl.BlockSpec((RR, BM, DD), lambda i: (0, i, 0)),
            pl.BlockSpec((DD, DD), lambda i: (0, 0)),
            pl.BlockSpec((DD, 1), lambda i: (0, 0)),
            pl.BlockSpec((DD, DD), lambda i: (0, 0)),
            pl.BlockSpec((DD, 1), lambda i: (0, 0)),
        ],
        out_specs=pl.BlockSpec((BM, DD), lambda i: (i, 0)),
        out_shape=jax.ShapeDtypeStruct((NN, DD), jnp.float32),
    )(y1, y2, um, uv, im, iv)


def kernel(adj_indices, adj_values, user_emb, item_emb,
           user_att_mat, user_att, item_att_mat, item_att):
    dsts = [adj_indices[r, 0] for r in range(RR)]
    srcs = [adj_indices[r, 1] for r in range(RR)]
    vals = [adj_values[r] for r in range(RR)]
    x0 = jnp.concatenate(
        [user_emb, item_emb,
         jnp.zeros((RR, NPAD - NN, DD), jnp.float32)], axis=1)
    y1 = _spmm(dsts, srcs, vals, x0)
    y2 = _spmm(dsts, srcs, vals, y1)
    return _mix(y1, y2, user_att_mat, user_att, item_att_mat, item_att)


# speedup vs baseline: 1.0553x; 1.0553x over previous
import jax
import jax.numpy as jnp
from jax import lax
from jax.experimental import pallas as pl
from jax.experimental.pallas import tpu as pltpu
from jax.experimental.pallas import tpu_sc as plsc

USER_N = 40000
ITEM_N = 60000
NN = USER_N + ITEM_N
NPAD = 102400
DD = 64
RR = 5
EE = 320000
NC = 2
NS = 16
EPT = EE // NS
CHUNK = 6400
KPC = NPAD // CHUNK // NC
STRIPE = CHUNK // NS
ZROWS = 80
BATCH = 128


def _spmm_body(*refs):
    dsts = refs[0:RR]
    srcs = refs[RR:2 * RR]
    vals = refs[2 * RR:3 * RR]
    x_hbm = refs[3 * RR]
    y_hbm = refs[3 * RR + 1]
    (dst_v, src_v, val_v, eidx_v,
     rows_a, gidx_a, sidx_a, vstg_a, sem_a, ssem_a,
     rows_b, gidx_b, sidx_b, vstg_b, sem_b, ssem_b,
     zidx_v, cnt_ref, acc_sh) = refs[3 * RR + 2:]

    core = lax.axis_index("c")
    tile = lax.axis_index("s")
    ebase = tile * EPT

    for r in range(RR):
        pltpu.sync_copy(dsts[r].at[pl.ds(ebase, EPT)], dst_v)
        pltpu.sync_copy(srcs[r].at[pl.ds(ebase, EPT)], src_v)
        pltpu.sync_copy(vals[r].at[pl.ds(ebase, EPT)], val_v)

        def chunk_body(k, carry, r=r):
            lo = (core * KPC + k) * CHUNK

            def zrow(i, c):
                for j in range(DD // 16):
                    rows_a[i, pl.ds(j * 16, 16)] = jnp.zeros((16,), jnp.float32)
                return c
            lax.fori_loop(0, ZROWS, zrow, 0)

            def zblk(z, c):
                zb = tile * STRIPE + z * ZROWS
                for j in range(ZROWS // 16):
                    zidx_v[pl.ds(j * 16, 16)] = (
                        zb + j * 16 + lax.iota(jnp.int32, 16))
                pltpu.sync_copy(rows_a.at[pl.ds(0, ZROWS)], acc_sh.at[zidx_v])
                return c
            lax.fori_loop(0, STRIPE // ZROWS, zblk, 0)
            plsc.subcore_barrier()

            cnt_ref[pl.ds(0, 16)] = jnp.zeros((16,), jnp.int32)

            FU = 5

            def filt(i, c):
                ms, es, cs = [], [], []
                for u in range(FU):
                    d16 = dst_v[pl.ds((i * FU + u) * 16, 16)]
                    m = (d16 >= lo) & (d16 < lo + CHUNK)
                    ms.append(m)
                    es.append(lax.iota(jnp.int32, 16) + (i * FU + u) * 16)
                    cs.append(plsc.cumsum(m.astype(jnp.int32)))
                cntv = cnt_ref[pl.ds(0, 16)]
                for u in range(FU):
                    plsc.store_scatter(eidx_v, [cntv + cs[u] - 1], es[u],
                                       mask=ms[u])
                    cntv = cntv + jnp.broadcast_to(cs[u][15], (16,))
                cnt_ref[pl.ds(0, 16)] = cntv
                return c
            lax.fori_loop(0, EPT // (16 * FU), filt, 0)

            cntv = cnt_ref[pl.ds(0, 16)]
            nb = ((cntv + (BATCH - 1)) // BATCH)[0]

            def stage(b, gidx_X, sidx_X, vstg_X):
                base = b * BATCH
                for j in range(BATCH // 16):
                    pos = base + j * 16 + lax.iota(jnp.int32, 16)
                    valid = pos < cnt_ref[pl.ds(0, 16)]
                    e16 = plsc.load_gather(eidx_v, [jnp.where(valid, pos, 0)])
                    e16 = jnp.where(valid, e16, 0)
                    s16 = plsc.load_gather(src_v, [e16])
                    d16 = plsc.load_gather(dst_v, [e16]) - lo
                    v16 = plsc.load_gather(val_v, [e16])
                    gidx_X[pl.ds(j * 16, 16)] = jnp.where(valid, s16, 0)
                    sidx_X[pl.ds(j * 16, 16)] = jnp.where(valid, d16, 0)
                    vstg_X[pl.ds(j * 16, 16)] = jnp.where(valid, v16, 0.0)

            def gather(gidx_X, rows_X, sem_X, r=r):
                return pltpu.make_async_copy(
                    x_hbm.at[r].at[gidx_X], rows_X, sem_X)

            def scatter(rows_X, sidx_X, ssem_X):
                return pltpu.make_async_copy(
                    rows_X, acc_sh.at[sidx_X], ssem_X)

            def consume(rows_X, sidx_X, vstg_X, ssem_X):
                def scale(g, cc):
                    v16 = vstg_X[pl.ds(g * 16, 16)]
                    for l in range(16):
                        vb = jnp.broadcast_to(v16[l], (16,))
                        i = g * 16 + l
                        for j in range(DD // 16):
                            rows_X[i, pl.ds(j * 16, 16)] = (
                                rows_X[i, pl.ds(j * 16, 16)] * vb)
                    return cc
                lax.fori_loop(0, BATCH // 16, scale, 0)
                scatter(rows_X, sidx_X, ssem_X).start(add=True)

            @pl.when(nb > 0)
            def _prologue():
                stage(0, gidx_a, sidx_a, vstg_a)
                gather(gidx_a, rows_a, sem_a).start()

            def pair_body(p, c):
                b0 = 2 * p

                @pl.when(b0 + 1 < nb)
                def _prefetch_b():
                    @pl.when(p > 0)
                    def _drain_b():
                        scatter(rows_b, sidx_b, ssem_b).wait()
                    stage(b0 + 1, gidx_b, sidx_b, vstg_b)
                    gather(gidx_b, rows_b, sem_b).start()

                gather(gidx_a, rows_a, sem_a).wait()
                consume(rows_a, sidx_a, vstg_a, ssem_a)

                @pl.when(b0 + 1 < nb)
                def _odd_batch():
                    @pl.when(b0 + 2 < nb)
                    def _prefetch_a():
                        scatter(rows_a, sidx_a, ssem_a).wait()
                        stage(b0 + 2, gidx_a, sidx_a, vstg_a)
                        gather(gidx_a, rows_a, sem_a).start()

                    gather(gidx_b, rows_b, sem_b).wait()
                    consume(rows_b, sidx_b, vstg_b, ssem_b)
                return c
            lax.fori_loop(0, (nb + 1) // 2, pair_body, 0)

            @pl.when(nb > 0)
            def _drain_last_a():
                scatter(rows_a, sidx_a, ssem_a).wait()

            @pl.when(nb >= 2)
            def _drain_last_b():
                scatter(rows_b, sidx_b, ssem_b).wait()
            plsc.subcore_barrier()

            def wblk(z, c, r=r):
                off = tile * STRIPE + z * ZROWS
                for j in range(ZROWS // 16):
                    zidx_v[pl.ds(j * 16, 16)] = (
                        off + j * 16 + lax.iota(jnp.int32, 16))
                pltpu.sync_copy(acc_sh.at[zidx_v], rows_a.at[pl.ds(0, ZROWS)])
                pltpu.sync_copy(rows_a.at[pl.ds(0, ZROWS)],
                                y_hbm.at[r].at[pl.ds(lo + off, ZROWS)])
                return c
            lax.fori_loop(0, STRIPE // ZROWS, wblk, 0)
            return carry
        lax.fori_loop(0, KPC, chunk_body, 0)


@jax.jit
def _spmm(dsts, srcs, vals, x):
    mesh = plsc.VectorSubcoreMesh(core_axis_name="c", subcore_axis_name="s",
                                  num_cores=NC, num_subcores=NS)
    return pl.kernel(
        _spmm_body,
        out_type=jax.ShapeDtypeStruct((RR, NPAD, DD), jnp.float32),
        mesh=mesh,
        compiler_params=pltpu.CompilerParams(
            needs_layout_passes=False, use_tc_tiling_on_sc=False),
        scratch_types=[
            pltpu.VMEM((EPT,), jnp.int32),
            pltpu.VMEM((EPT,), jnp.int32),
            pltpu.VMEM((EPT,), jnp.float32),
            pltpu.VMEM((EPT,), jnp.int32),
            pltpu.VMEM((BATCH, DD), jnp.float32),
            pltpu.VMEM((BATCH,), jnp.int32),
            pltpu.VMEM((BATCH,), jnp.int32),
            pltpu.VMEM((BATCH,), jnp.float32),
            pltpu.SemaphoreType.DMA,
            pltpu.SemaphoreType.DMA,
            pltpu.VMEM((BATCH, DD), jnp.float32),
            pltpu.VMEM((BATCH,), jnp.int32),
            pltpu.VMEM((BATCH,), jnp.int32),
            pltpu.VMEM((BATCH,), jnp.float32),
            pltpu.SemaphoreType.DMA,
            pltpu.SemaphoreType.DMA,
            pltpu.VMEM((ZROWS,), jnp.int32),
            pltpu.VMEM((16,), jnp.int32),
            pltpu.VMEM_SHARED((CHUNK, DD), jnp.float32),
        ],
    )(*dsts, *srcs, *vals, x)


BM = 2000
UBLOCKS = USER_N // BM


def _mix_body(y1_ref, y2_ref, um_ref, uv_ref, im_ref, iv_ref, o_ref):
    i = pl.program_id(0)
    is_user = i < UBLOCKS
    mat = jnp.where(is_user, um_ref[...], im_ref[...])
    vec = jnp.where(is_user, uv_ref[...], iv_ref[...])
    ms, ws = [], []
    for r in range(RR):
        m = (y1_ref[r] + y2_ref[r]) * 0.5
        h = jnp.dot(m, mat, preferred_element_type=jnp.float32)
        w = jnp.tanh(jnp.dot(h, vec, preferred_element_type=jnp.float32))
        ms.append(m)
        ws.append(w)
    wmax = ws[0]
    for r in range(1, RR):
        wmax = jnp.maximum(wmax, ws[r])
    es = [jnp.exp(w - wmax) for w in ws]
    denom = es[0]
    for r in range(1, RR):
        denom = denom + es[r]
    acc = ms[0] * (es[0] / denom)
    for r in range(1, RR):
        acc = acc + ms[r] * (es[r] / denom)
    o_ref[...] = acc


@jax.jit
def _mix(y1, y2, um, uv, im, iv):
    return pl.pallas_call(
        _mix_body,
        grid=(NN // BM,),
        in_specs=[
            pl.BlockSpec((RR, BM, DD), lambda i: (0, i, 0)),
            pl.BlockSpec((RR, BM, DD), lambda i: (0, i, 0)),
            pl.BlockSpec((DD, DD), lambda i: (0, 0)),
            pl.BlockSpec((DD, 1), lambda i: (0, 0)),
            pl.BlockSpec((DD, DD), lambda i: (0, 0)),
            pl.BlockSpec((DD, 1), lambda i: (0, 0)),
        ],
        out_specs=pl.BlockSpec((BM, DD), lambda i: (i, 0)),
        out_shape=jax.ShapeDtypeStruct((NN, DD), jnp.float32),
    )(y1, y2, um, uv, im, iv)


def kernel(adj_indices, adj_values, user_emb, item_emb,
           user_att_mat, user_att, item_att_mat, item_att):
    dsts = [adj_indices[r, 0] for r in range(RR)]
    srcs = [adj_indices[r, 1] for r in range(RR)]
    vals = [adj_values[r] for r in range(RR)]
    x0 = jnp.concatenate(
        [user_emb, item_emb,
         jnp.zeros((RR, NPAD - NN, DD), jnp.float32)], axis=1)
    y1 = _spmm(dsts, srcs, vals, x0)
    y2 = _spmm(dsts, srcs, vals, y1)
    return _mix(y1, y2, user_att_mat, user_att, item_att_mat, item_att)
